# baseline (speedup 1.0000x reference)
import jax
import jax.numpy as jnp
from jax.experimental import pallas as pl
from jax.experimental.pallas import tpu as pltpu

_COLS = (0, 1, 3, 7, 15, 31, 50)
_KS = (4096, 2048, 1024, 512, 256, 128, 64)
_BASES = (0, 4096, 6144, 7168, 7680, 7936, 8064)
_N = 8192
_B = 32
_INT32_MAX = 0x7FFFFFFF


def _sortable(v):
    b = jax.lax.bitcast_convert_type(v + 0.0, jnp.int32)
    return b ^ (jnp.right_shift(b, 31) & _INT32_MAX)


def _gather_row(src, idx):
    lane = idx & 127
    blk = jnp.right_shift(idx, 7)
    out = None
    for b in range(64):
        part = jnp.take_along_axis(src[:, b * 128:(b + 1) * 128], lane, axis=1)
        out = part if out is None else jnp.where(blk == b, part, out)
    return out


def _small_stage(k, i, p, q, cidx):
    ch = k.shape[1]
    chbits = ch.bit_length() - 1
    d = 1 << q
    iota = jax.lax.broadcasted_iota(jnp.int32, (_B, ch), 1)
    pat = jax.lax.broadcasted_iota(jnp.int32, (_B, 128), 1) ^ d

    def g(x):
        if ch == 128:
            return jnp.take_along_axis(x, pat, axis=1)
        return jnp.concatenate(
            [
                jnp.take_along_axis(x[:, c * 128:(c + 1) * 128], pat, axis=1)
                for c in range(ch // 128)
            ],
            axis=1,
        )

    kp, ip = g(k), g(i)
    bq = jnp.right_shift(iota, q) & 1
    if p + 1 < chbits:
        bk = jnp.right_shift(iota, p + 1) & 1
    else:
        bk = jnp.right_shift(cidx, p + 1 - chbits) & 1
    ts = (bq ^ bk) == 0
    ps = (kp < k) | ((kp == k) & (ip < i))
    tp = ps == ts
    return jnp.where(tp, kp, k), jnp.where(tp, ip, i)


def _big_stage(key, idx, n, p, q):
    d = 1 << q
    outs_k, outs_i = [], []
    for j in range(n // (2 * d)):
        o = j * 2 * d
        ka, kb = key[:, o:o + d], key[:, o + d:o + 2 * d]
        ia, ib = idx[:, o:o + d], idx[:, o + d:o + 2 * d]
        a_sm = (ka < kb) | ((ka == kb) & (ia < ib))
        asc = ((j >> (p - q)) & 1) == 0
        take_b = ~a_sm if asc else a_sm
        outs_k += [jnp.where(take_b, kb, ka), jnp.where(take_b, ka, kb)]
        outs_i += [jnp.where(take_b, ib, ia), jnp.where(take_b, ia, ib)]
    return jnp.concatenate(outs_k, 1), jnp.concatenate(outs_i, 1)


def _sort_ref(kref, iref, n):
    nbits = n.bit_length() - 1
    ch = min(n, 1024)

    def chunk_a(c, _):
        o = pl.multiple_of(c * ch, 128)
        k = kref[:, pl.ds(o, ch)]
        i = iref[:, pl.ds(o, ch)]
        for p in range(min(nbits, 7)):
            for q in range(p, -1, -1):
                k, i = _small_stage(k, i, p, q, c)
        kref[:, pl.ds(o, ch)] = k
        iref[:, pl.ds(o, ch)] = i
        return 0

    jax.lax.fori_loop(0, max(n // ch, 1), chunk_a, 0, unroll=4)

    for p in range(7, nbits):
        key = kref[:, :n]
        idx = iref[:, :n]
        for q in range(p, 6, -1):
            key, idx = _big_stage(key, idx, n, p, q)
        kref[:, :n] = key
        iref[:, :n] = idx

        def chunk_b(c, _, p=p):
            o = pl.multiple_of(c * ch, 128)
            k = kref[:, pl.ds(o, ch)]
            i = iref[:, pl.ds(o, ch)]
            for q in range(6, -1, -1):
                k, i = _small_stage(k, i, p, q, c)
            kref[:, pl.ds(o, ch)] = k
            iref[:, pl.ds(o, ch)] = i
            return 0

        jax.lax.fori_loop(0, n // ch, chunk_b, 0, unroll=4)


def _sh_kernel(cols_ref, out_ref, kref, iref):
    iota = jax.lax.broadcasted_iota(jnp.int32, (_B, _N), 1)
    kref[:, :] = _sortable(cols_ref[0])
    iref[:, :] = iota
    for r in range(7):
        n = _N >> r
        _sort_ref(kref, iref, n)
        k, base = _KS[r], _BASES[r]
        idx = iref[:, :n]
        if r < 6:
            out_ref[:, base:base + k] = idx[:, :k].astype(jnp.float32)
            surv = idx[:, k:]
            vals = _gather_row(cols_ref[r + 1], surv)
            kref[:, :n - k] = _sortable(vals)
            iref[:, :n - k] = surv
        else:
            out_ref[:, base:] = idx[:, :128].astype(jnp.float32)


def kernel(learning_curves, mask):
    del mask
    cols = jnp.transpose(
        learning_curves[:, :, jnp.array(_COLS)], (2, 0, 1)
    )
    return pl.pallas_call(
        _sh_kernel,
        out_shape=jax.ShapeDtypeStruct((_B, _N), jnp.float32),
        scratch_shapes=[
            pltpu.VMEM((_B, _N), jnp.int32),
            pltpu.VMEM((_B, _N), jnp.int32),
        ],
    )(cols)


# speedup vs baseline: 1.1812x; 1.1812x over previous
import jax
import jax.numpy as jnp
from jax.experimental import pallas as pl
from jax.experimental.pallas import tpu as pltpu

_COLS = (0, 1, 3, 7, 15, 31, 50)
_KS = (4096, 2048, 1024, 512, 256, 128, 64)
_BASES = (0, 4096, 6144, 7168, 7680, 7936, 8064)
_N = 8192
_B = 32
_INT32_MAX = 0x7FFFFFFF


def _sortable(v):
    b = jax.lax.bitcast_convert_type(v + 0.0, jnp.int32)
    return b ^ (jnp.right_shift(b, 31) & _INT32_MAX)


def _gather_row(src, idx):
    lane = idx & 127
    blk = jnp.right_shift(idx, 7)
    out = None
    for b in range(64):
        part = jnp.take_along_axis(src[:, b * 128:(b + 1) * 128], lane, axis=1)
        out = part if out is None else jnp.where(blk == b, part, out)
    return out


def _small_stage(k, i, p, q, cidx):
    ch = k.shape[1]
    chbits = ch.bit_length() - 1
    d = 1 << q
    iota = jax.lax.broadcasted_iota(jnp.int32, (_B, ch), 1)
    pat = jax.lax.broadcasted_iota(jnp.int32, (_B, 128), 1) ^ d

    def g(x):
        if ch == 128:
            return jnp.take_along_axis(x, pat, axis=1)
        return jnp.concatenate(
            [
                jnp.take_along_axis(x[:, c * 128:(c + 1) * 128], pat, axis=1)
                for c in range(ch // 128)
            ],
            axis=1,
        )

    kp, ip = g(k), g(i)
    bq = jnp.right_shift(iota, q) & 1
    if p + 1 < chbits:
        bk = jnp.right_shift(iota, p + 1) & 1
    else:
        bk = jnp.right_shift(cidx, p + 1 - chbits) & 1
    ts = (bq ^ bk) == 0
    ps = (kp < k) | ((kp == k) & (ip < i))
    tp = ps == ts
    return jnp.where(tp, kp, k), jnp.where(tp, ip, i)


def _big_stage(key, idx, n, p, q):
    d = 1 << q
    outs_k, outs_i = [], []
    for j in range(n // (2 * d)):
        o = j * 2 * d
        ka, kb = key[:, o:o + d], key[:, o + d:o + 2 * d]
        ia, ib = idx[:, o:o + d], idx[:, o + d:o + 2 * d]
        a_sm = (ka < kb) | ((ka == kb) & (ia < ib))
        asc = ((j >> (p - q)) & 1) == 0
        take_b = ~a_sm if asc else a_sm
        outs_k += [jnp.where(take_b, kb, ka), jnp.where(take_b, ka, kb)]
        outs_i += [jnp.where(take_b, ib, ia), jnp.where(take_b, ia, ib)]
    return jnp.concatenate(outs_k, 1), jnp.concatenate(outs_i, 1)


def _sort_ref(kref, iref, n):
    nbits = n.bit_length() - 1
    ch = min(n, 2048)

    def chunk_a(c, _):
        o = pl.multiple_of(c * ch, 128)
        k = kref[:, pl.ds(o, ch)]
        i = iref[:, pl.ds(o, ch)]
        for p in range(min(nbits, 7)):
            for q in range(p, -1, -1):
                k, i = _small_stage(k, i, p, q, c)
        kref[:, pl.ds(o, ch)] = k
        iref[:, pl.ds(o, ch)] = i
        return 0

    jax.lax.fori_loop(0, max(n // ch, 1), chunk_a, 0, unroll=False)

    for p in range(7, nbits):
        key = kref[:, :n]
        idx = iref[:, :n]
        for q in range(p, 6, -1):
            key, idx = _big_stage(key, idx, n, p, q)
        kref[:, :n] = key
        iref[:, :n] = idx

        def chunk_b(c, _, p=p):
            o = pl.multiple_of(c * ch, 128)
            k = kref[:, pl.ds(o, ch)]
            i = iref[:, pl.ds(o, ch)]
            for q in range(6, -1, -1):
                k, i = _small_stage(k, i, p, q, c)
            kref[:, pl.ds(o, ch)] = k
            iref[:, pl.ds(o, ch)] = i
            return 0

        jax.lax.fori_loop(0, n // ch, chunk_b, 0, unroll=False)


def _sh_kernel(cols_ref, out_ref, kref, iref):
    iota = jax.lax.broadcasted_iota(jnp.int32, (_B, _N), 1)
    kref[:, :] = _sortable(cols_ref[0])
    iref[:, :] = iota
    for r in range(7):
        n = _N >> r
        _sort_ref(kref, iref, n)
        k, base = _KS[r], _BASES[r]
        idx = iref[:, :n]
        if r < 6:
            out_ref[:, base:base + k] = idx[:, :k].astype(jnp.float32)
            surv = idx[:, k:]
            vals = _gather_row(cols_ref[r + 1], surv)
            kref[:, :n - k] = _sortable(vals)
            iref[:, :n - k] = surv
        else:
            out_ref[:, base:] = idx[:, :128].astype(jnp.float32)


def kernel(learning_curves, mask):
    del mask
    cols = jnp.transpose(
        learning_curves[:, :, jnp.array(_COLS)], (2, 0, 1)
    )
    return pl.pallas_call(
        _sh_kernel,
        out_shape=jax.ShapeDtypeStruct((_B, _N), jnp.float32),
        scratch_shapes=[
            pltpu.VMEM((_B, _N), jnp.int32),
            pltpu.VMEM((_B, _N), jnp.int32),
        ],
    )(cols)


# speedup vs baseline: 1.2024x; 1.0180x over previous
import jax
import jax.numpy as jnp
from jax.experimental import pallas as pl
from jax.experimental.pallas import tpu as pltpu

_COLS = (0, 1, 3, 7, 15, 31, 50)
_KS = (4096, 2048, 1024, 512, 256, 128, 64)
_BASES = (0, 4096, 6144, 7168, 7680, 7936, 8064)
_N = 8192
_B = 32
_INT32_MAX = 0x7FFFFFFF


def _sortable(v):
    b = jax.lax.bitcast_convert_type(v + 0.0, jnp.int32)
    return b ^ (jnp.right_shift(b, 31) & _INT32_MAX)


def _gather_row(src, idx):
    lane = idx & 127
    blk = jnp.right_shift(idx, 7)
    out = None
    for b in range(64):
        part = jnp.take_along_axis(src[:, b * 128:(b + 1) * 128], lane, axis=1)
        out = part if out is None else jnp.where(blk == b, part, out)
    return out


def _small_stage(k, i, p, q, cidx):
    ch = k.shape[1]
    chbits = ch.bit_length() - 1
    d = 1 << q
    iota = jax.lax.broadcasted_iota(jnp.int32, (_B, ch), 1)
    pat = jax.lax.broadcasted_iota(jnp.int32, (_B, 128), 1) ^ d

    def g(x):
        if ch == 128:
            return jnp.take_along_axis(x, pat, axis=1)
        return jnp.concatenate(
            [
                jnp.take_along_axis(x[:, c * 128:(c + 1) * 128], pat, axis=1)
                for c in range(ch // 128)
            ],
            axis=1,
        )

    kp, ip = g(k), g(i)
    bq = jnp.right_shift(iota, q) & 1
    if p + 1 < chbits:
        bk = jnp.right_shift(iota, p + 1) & 1
    else:
        bk = jnp.right_shift(cidx, p + 1 - chbits) & 1
    ts = (bq ^ bk) == 0
    ps = (kp < k) | ((kp == k) & (ip < i))
    tp = ps == ts
    return jnp.where(tp, kp, k), jnp.where(tp, ip, i)


def _big_stage(key, idx, n, p, q):
    d = 1 << q
    outs_k, outs_i = [], []
    for j in range(n // (2 * d)):
        o = j * 2 * d
        ka, kb = key[:, o:o + d], key[:, o + d:o + 2 * d]
        ia, ib = idx[:, o:o + d], idx[:, o + d:o + 2 * d]
        a_sm = (ka < kb) | ((ka == kb) & (ia < ib))
        asc = ((j >> (p - q)) & 1) == 0
        take_b = ~a_sm if asc else a_sm
        outs_k += [jnp.where(take_b, kb, ka), jnp.where(take_b, ka, kb)]
        outs_i += [jnp.where(take_b, ib, ia), jnp.where(take_b, ia, ib)]
    return jnp.concatenate(outs_k, 1), jnp.concatenate(outs_i, 1)


def _sort_ref(kref, iref, n):
    nbits = n.bit_length() - 1
    ch = n

    def chunk_a(c, _):
        o = pl.multiple_of(c * ch, 128)
        k = kref[:, pl.ds(o, ch)]
        i = iref[:, pl.ds(o, ch)]
        for p in range(min(nbits, 7)):
            for q in range(p, -1, -1):
                k, i = _small_stage(k, i, p, q, c)
        kref[:, pl.ds(o, ch)] = k
        iref[:, pl.ds(o, ch)] = i
        return 0

    jax.lax.fori_loop(0, max(n // ch, 1), chunk_a, 0, unroll=False)

    for p in range(7, nbits):
        key = kref[:, :n]
        idx = iref[:, :n]
        for q in range(p, 6, -1):
            key, idx = _big_stage(key, idx, n, p, q)
        kref[:, :n] = key
        iref[:, :n] = idx

        def chunk_b(c, _, p=p):
            o = pl.multiple_of(c * ch, 128)
            k = kref[:, pl.ds(o, ch)]
            i = iref[:, pl.ds(o, ch)]
            for q in range(6, -1, -1):
                k, i = _small_stage(k, i, p, q, c)
            kref[:, pl.ds(o, ch)] = k
            iref[:, pl.ds(o, ch)] = i
            return 0

        jax.lax.fori_loop(0, n // ch, chunk_b, 0, unroll=False)


def _sh_kernel(cols_ref, out_ref, kref, iref):
    iota = jax.lax.broadcasted_iota(jnp.int32, (_B, _N), 1)
    kref[:, :] = _sortable(cols_ref[0])
    iref[:, :] = iota
    for r in range(7):
        n = _N >> r
        _sort_ref(kref, iref, n)
        k, base = _KS[r], _BASES[r]
        idx = iref[:, :n]
        if r < 6:
            out_ref[:, base:base + k] = idx[:, :k].astype(jnp.float32)
            surv = idx[:, k:]
            vals = _gather_row(cols_ref[r + 1], surv)
            kref[:, :n - k] = _sortable(vals)
            iref[:, :n - k] = surv
        else:
            out_ref[:, base:] = idx[:, :128].astype(jnp.float32)


def kernel(learning_curves, mask):
    del mask
    cols = jnp.transpose(
        learning_curves[:, :, jnp.array(_COLS)], (2, 0, 1)
    )
    return pl.pallas_call(
        _sh_kernel,
        out_shape=jax.ShapeDtypeStruct((_B, _N), jnp.float32),
        scratch_shapes=[
            pltpu.VMEM((_B, _N), jnp.int32),
            pltpu.VMEM((_B, _N), jnp.int32),
        ],
    )(cols)


# speedup vs baseline: 1.2549x; 1.0436x over previous
import jax
import jax.numpy as jnp
from jax.experimental import pallas as pl
from jax.experimental.pallas import tpu as pltpu

_COLS = (0, 1, 3, 7, 15, 31, 50)
_KS = (4096, 2048, 1024, 512, 256, 128, 64)
_BASES = (0, 4096, 6144, 7168, 7680, 7936, 8064)
_N = 8192
_B = 32
_INT32_MAX = 0x7FFFFFFF


def _sortable(v):
    b = jax.lax.bitcast_convert_type(v + 0.0, jnp.int32)
    return b ^ (jnp.right_shift(b, 31) & _INT32_MAX)


def _gather_row(src, idx):
    lane = idx & 127
    blk = jnp.right_shift(idx, 7)
    out = None
    for b in range(64):
        part = jnp.take_along_axis(src[:, b * 128:(b + 1) * 128], lane, axis=1)
        out = part if out is None else jnp.where(blk == b, part, out)
    return out


def _small_stage(k, i, p, q, cidx):
    ch = k.shape[1]
    chbits = ch.bit_length() - 1
    d = 1 << q
    iota = jax.lax.broadcasted_iota(jnp.int32, (_B, ch), 1)
    pat = jax.lax.broadcasted_iota(jnp.int32, (_B, 128), 1) ^ d

    def g(x):
        if ch == 128:
            return jnp.take_along_axis(x, pat, axis=1)
        return jnp.concatenate(
            [
                jnp.take_along_axis(x[:, c * 128:(c + 1) * 128], pat, axis=1)
                for c in range(ch // 128)
            ],
            axis=1,
        )

    kp, ip = g(k), g(i)
    bq = jnp.right_shift(iota, q) & 1
    if p + 1 < chbits:
        bk = jnp.right_shift(iota, p + 1) & 1
    else:
        bk = jnp.right_shift(cidx, p + 1 - chbits) & 1
    ts = (bq ^ bk) == 0
    ps = (kp < k) | ((kp == k) & (ip < i))
    tp = ps == ts
    return jnp.where(tp, kp, k), jnp.where(tp, ip, i)


def _big_stage(key, idx, n, p, q):
    d = 1 << q
    outs_k, outs_i = [], []
    for j in range(n // (2 * d)):
        o = j * 2 * d
        ka, kb = key[:, o:o + d], key[:, o + d:o + 2 * d]
        ia, ib = idx[:, o:o + d], idx[:, o + d:o + 2 * d]
        a_sm = (ka < kb) | ((ka == kb) & (ia < ib))
        asc = ((j >> (p - q)) & 1) == 0
        take_b = ~a_sm if asc else a_sm
        outs_k += [jnp.where(take_b, kb, ka), jnp.where(take_b, ka, kb)]
        outs_i += [jnp.where(take_b, ib, ia), jnp.where(take_b, ia, ib)]
    return jnp.concatenate(outs_k, 1), jnp.concatenate(outs_i, 1)


def _sort_val(key, idx, full):
    n = key.shape[1]
    nbits = n.bit_length() - 1

    for p in range(min(nbits, 7)):
        for q in range(p, -1, -1):
            key, idx = _small_stage(key, idx, p, q, 0)

    for p in range(7, nbits):
        if not full and p == nbits - 1:
            key, idx = _big_stage(key, idx, n, p, p)
            kl, il = key[:, :n // 2], idx[:, :n // 2]
            for q in range(p - 1, 6, -1):
                kl, il = _big_stage(kl, il, n // 2, p, q)
            for q in range(6, -1, -1):
                kl, il = _small_stage(kl, il, p, q, 0)
            return kl, il, idx[:, n // 2:]
        for q in range(p, 6, -1):
            key, idx = _big_stage(key, idx, n, p, q)
        for q in range(6, -1, -1):
            key, idx = _small_stage(key, idx, p, q, 0)
    return key, idx


def _sh_kernel(cols_ref, out_ref):
    idx = jax.lax.broadcasted_iota(jnp.int32, (_B, _N), 1)
    key = _sortable(cols_ref[0])
    for r in range(7):
        k, base = _KS[r], _BASES[r]
        if r < 6:
            _, dead, surv = _sort_val(key, idx, full=False)
            out_ref[:, base:base + k] = dead.astype(jnp.float32)
            vals = _gather_row(cols_ref[r + 1], surv)
            key = _sortable(vals)
            idx = surv
        else:
            _, idx = _sort_val(key, idx, full=True)
            out_ref[:, base:] = idx.astype(jnp.float32)


def kernel(learning_curves, mask):
    del mask
    cols = jnp.transpose(
        learning_curves[:, :, jnp.array(_COLS)], (2, 0, 1)
    )
    return pl.pallas_call(
        _sh_kernel,
        out_shape=jax.ShapeDtypeStruct((_B, _N), jnp.float32),
    )(cols)


# speedup vs baseline: 1.2563x; 1.0011x over previous
import jax
import jax.numpy as jnp
from jax.experimental import pallas as pl
from jax.experimental.pallas import tpu as pltpu

_COLS = (0, 1, 3, 7, 15, 31, 50)
_KS = (4096, 2048, 1024, 512, 256, 128, 64)
_BASES = (0, 4096, 6144, 7168, 7680, 7936, 8064)
_N = 8192
_B = 32
_INT32_MAX = 0x7FFFFFFF


def _sortable(v):
    b = jax.lax.bitcast_convert_type(v + 0.0, jnp.int32)
    return b ^ (jnp.right_shift(b, 31) & _INT32_MAX)


def _gather_row(src, idx):
    lane = idx & 127
    blk = jnp.right_shift(idx, 7)
    parts = [
        jnp.take_along_axis(src[:, b * 128:(b + 1) * 128], lane, axis=1)
        for b in range(64)
    ]
    for bit in range(6):
        sel = (blk & (1 << bit)) != 0
        parts = [
            jnp.where(sel, parts[2 * i + 1], parts[2 * i])
            for i in range(len(parts) // 2)
        ]
    return parts[0]


def _small_stage(k, i, p, q, cidx):
    ch = k.shape[1]
    chbits = ch.bit_length() - 1
    d = 1 << q
    iota = jax.lax.broadcasted_iota(jnp.int32, (_B, ch), 1)
    pat = jax.lax.broadcasted_iota(jnp.int32, (_B, 128), 1) ^ d

    def g(x):
        if ch == 128:
            return jnp.take_along_axis(x, pat, axis=1)
        return jnp.concatenate(
            [
                jnp.take_along_axis(x[:, c * 128:(c + 1) * 128], pat, axis=1)
                for c in range(ch // 128)
            ],
            axis=1,
        )

    kp, ip = g(k), g(i)
    bq = jnp.right_shift(iota, q) & 1
    if p + 1 < chbits:
        bk = jnp.right_shift(iota, p + 1) & 1
    else:
        bk = jnp.right_shift(cidx, p + 1 - chbits) & 1
    ts = (bq ^ bk) == 0
    ps = (kp < k) | ((kp == k) & (ip < i))
    tp = ps == ts
    return jnp.where(tp, kp, k), jnp.where(tp, ip, i)


def _big_stage(key, idx, n, p, q):
    d = 1 << q
    outs_k, outs_i = [], []
    for j in range(n // (2 * d)):
        o = j * 2 * d
        ka, kb = key[:, o:o + d], key[:, o + d:o + 2 * d]
        ia, ib = idx[:, o:o + d], idx[:, o + d:o + 2 * d]
        a_sm = (ka < kb) | ((ka == kb) & (ia < ib))
        asc = ((j >> (p - q)) & 1) == 0
        take_b = ~a_sm if asc else a_sm
        outs_k += [jnp.where(take_b, kb, ka), jnp.where(take_b, ka, kb)]
        outs_i += [jnp.where(take_b, ib, ia), jnp.where(take_b, ia, ib)]
    return jnp.concatenate(outs_k, 1), jnp.concatenate(outs_i, 1)


def _sort_val(key, idx, full):
    n = key.shape[1]
    nbits = n.bit_length() - 1

    for p in range(min(nbits, 7)):
        for q in range(p, -1, -1):
            key, idx = _small_stage(key, idx, p, q, 0)

    for p in range(7, nbits):
        if not full and p == nbits - 1:
            key, idx = _big_stage(key, idx, n, p, p)
            kl, il = key[:, :n // 2], idx[:, :n // 2]
            for q in range(p - 1, 6, -1):
                kl, il = _big_stage(kl, il, n // 2, p, q)
            for q in range(6, -1, -1):
                kl, il = _small_stage(kl, il, p, q, 0)
            return kl, il, idx[:, n // 2:]
        for q in range(p, 6, -1):
            key, idx = _big_stage(key, idx, n, p, q)
        for q in range(6, -1, -1):
            key, idx = _small_stage(key, idx, p, q, 0)
    return key, idx


def _sh_kernel(cols_ref, out_ref):
    idx = jax.lax.broadcasted_iota(jnp.int32, (_B, _N), 1)
    key = _sortable(cols_ref[0])
    for r in range(7):
        k, base = _KS[r], _BASES[r]
        if r < 6:
            _, dead, surv = _sort_val(key, idx, full=False)
            out_ref[:, base:base + k] = dead.astype(jnp.float32)
            vals = _gather_row(cols_ref[r + 1], surv)
            key = _sortable(vals)
            idx = surv
        else:
            _, idx = _sort_val(key, idx, full=True)
            out_ref[:, base:] = idx.astype(jnp.float32)


def kernel(learning_curves, mask):
    del mask
    cols = jnp.transpose(
        learning_curves[:, :, jnp.array(_COLS)], (2, 0, 1)
    )
    return pl.pallas_call(
        _sh_kernel,
        out_shape=jax.ShapeDtypeStruct((_B, _N), jnp.float32),
    )(cols)


# speedup vs baseline: 1.2563x; 1.0000x over previous
import jax
import jax.numpy as jnp
from jax.experimental import pallas as pl

_COLS = (0, 1, 3, 7, 15, 31, 50)
_KS = (4096, 2048, 1024, 512, 256, 128, 64)
_BASES = (0, 4096, 6144, 7168, 7680, 7936, 8064)
_N = 8192
_B = 32
_INT32_MAX = 0x7FFFFFFF


def _sortable(v):
    b = jax.lax.bitcast_convert_type(v + 0.0, jnp.int32)
    return b ^ (jnp.right_shift(b, 31) & _INT32_MAX)


def _gather_row(src, idx):
    lane = idx & 127
    blk = jnp.right_shift(idx, 7)
    parts = [
        jnp.take_along_axis(src[:, b * 128:(b + 1) * 128], lane, axis=1)
        for b in range(64)
    ]
    for bit in range(6):
        sel = (blk & (1 << bit)) != 0
        parts = [
            jnp.where(sel, parts[2 * i + 1], parts[2 * i])
            for i in range(len(parts) // 2)
        ]
    return parts[0]


def _small_stage(k, i, p, q, cidx):
    ch = k.shape[1]
    chbits = ch.bit_length() - 1
    d = 1 << q
    iota = jax.lax.broadcasted_iota(jnp.int32, (_B, ch), 1)
    pat = jax.lax.broadcasted_iota(jnp.int32, (_B, 128), 1) ^ d

    def g(x):
        if ch == 128:
            return jnp.take_along_axis(x, pat, axis=1)
        return jnp.concatenate(
            [
                jnp.take_along_axis(x[:, c * 128:(c + 1) * 128], pat, axis=1)
                for c in range(ch // 128)
            ],
            axis=1,
        )

    kp, ip = g(k), g(i)
    bq = jnp.right_shift(iota, q) & 1
    if p + 1 < chbits:
        bk = jnp.right_shift(iota, p + 1) & 1
    else:
        bk = jnp.right_shift(cidx, p + 1 - chbits) & 1
    ts = (bq ^ bk) == 0
    ps = (kp < k) | ((kp == k) & (ip < i))
    tp = ps == ts
    return jnp.where(tp, kp, k), jnp.where(tp, ip, i)


def _big_stage(key, idx, n, p, q):
    d = 1 << q
    outs_k, outs_i = [], []
    for j in range(n // (2 * d)):
        o = j * 2 * d
        ka, kb = key[:, o:o + d], key[:, o + d:o + 2 * d]
        ia, ib = idx[:, o:o + d], idx[:, o + d:o + 2 * d]
        a_sm = (ka < kb) | ((ka == kb) & (ia < ib))
        asc = ((j >> (p - q)) & 1) == 0
        take_b = ~a_sm if asc else a_sm
        outs_k += [jnp.where(take_b, kb, ka), jnp.where(take_b, ka, kb)]
        outs_i += [jnp.where(take_b, ib, ia), jnp.where(take_b, ia, ib)]
    return jnp.concatenate(outs_k, 1), jnp.concatenate(outs_i, 1)


def _sort_val(key, idx, full):
    n = key.shape[1]
    nbits = n.bit_length() - 1

    for p in range(min(nbits, 7)):
        for q in range(p, -1, -1):
            key, idx = _small_stage(key, idx, p, q, 0)

    for p in range(7, nbits):
        if not full and p == nbits - 1:
            key, idx = _big_stage(key, idx, n, p, p)
            kl, il = key[:, :n // 2], idx[:, :n // 2]
            for q in range(p - 1, 6, -1):
                kl, il = _big_stage(kl, il, n // 2, p, q)
            for q in range(6, -1, -1):
                kl, il = _small_stage(kl, il, p, q, 0)
            return kl, il, idx[:, n // 2:]
        for q in range(p, 6, -1):
            key, idx = _big_stage(key, idx, n, p, q)
        for q in range(6, -1, -1):
            key, idx = _small_stage(key, idx, p, q, 0)
    return key, idx


def _sh_kernel(cols_ref, out_ref):
    idx = jax.lax.broadcasted_iota(jnp.int32, (_B, _N), 1)
    key = _sortable(cols_ref[0])
    for r in range(7):
        k, base = _KS[r], _BASES[r]
        if r < 6:
            _, dead, surv = _sort_val(key, idx, full=False)
            out_ref[:, base:base + k] = dead.astype(jnp.float32)
            vals = _gather_row(cols_ref[r + 1], surv)
            key = _sortable(vals)
            idx = surv
        else:
            _, idx = _sort_val(key, idx, full=True)
            out_ref[:, base:] = idx.astype(jnp.float32)


def kernel(learning_curves, mask):
    del mask
    cols = jnp.transpose(
        learning_curves[:, :, jnp.array(_COLS)], (2, 0, 1)
    )
    return pl.pallas_call(
        _sh_kernel,
        out_shape=jax.ShapeDtypeStruct((_B, _N), jnp.float32),
    )(cols)
